# baseline (speedup 1.0000x reference)
import functools

import jax
import jax.numpy as jnp
from jax.experimental import pallas as pl
from jax.experimental.pallas import tpu as pltpu

B, C, INTER, H, W = 8, 96, 96, 128, 128
K_CLS, K_REG = 9, 36
HW = H * W
ROWS_PER_CHUNK = 128
N_CHUNK = ROWS_PER_CHUNK * W
N_CHUNKS = H // ROWS_PER_CHUNK
PAD_ROWS = 2
HW_PAD = (H + 2 * PAD_ROWS) * W
REG_OFF = 16
M_SMALL = 56
B_PER_STEP = 1
K_BIG = 9 * C + 8


def _rpn_kernel(x_ref, w_all_ref, w_small_ref, bias_ref,
                cls_ref, reg_ref, s9_ref):
    w_all = w_all_ref[...]
    w_small = w_small_ref[...]
    b_small = bias_ref[0:M_SMALL, 0:1]
    lane = jax.lax.broadcasted_iota(jnp.int32, (C, HW), 1) & (W - 1)

    @pl.when(pl.program_id(0) == 0)
    def _init_const_rows():
        s9_ref[9 * C:9 * C + 1, :] = jnp.ones((1, HW), jnp.bfloat16)
        s9_ref[9 * C + 1:K_BIG, :] = jnp.zeros((7, HW), jnp.bfloat16)

    x = x_ref[0].astype(jnp.bfloat16).reshape(C, HW)
    xl = jnp.where(lane == 0, 0.0, jnp.roll(x, 1, axis=1))
    xr = jnp.where(lane == W - 1, 0.0, jnp.roll(x, -1, axis=1))
    zrow = jnp.zeros((C, W), jnp.bfloat16)
    for dx, v in ((0, xl), (1, x), (2, xr)):
        r0 = dx * C
        s9_ref[r0:r0 + C, 0:W] = zrow
        s9_ref[r0:r0 + C, pl.ds(W, HW - W)] = v[:, 0:HW - W]
        r1 = (3 + dx) * C
        s9_ref[r1:r1 + C, :] = v
        r2 = (6 + dx) * C
        s9_ref[r2:r2 + C, 0:HW - W] = v[:, W:HW]
        s9_ref[r2:r2 + C, pl.ds(HW - W, W)] = zrow

    for c in range(N_CHUNKS):
        acc = jnp.dot(w_all, s9_ref[:, pl.ds(c * N_CHUNK, N_CHUNK)],
                      preferred_element_type=jnp.float32)
        inter = jnp.maximum(acc, 0.0).astype(jnp.bfloat16)

        outs = jnp.dot(w_small, inter,
                       preferred_element_type=jnp.float32) + b_small
        cls_c = jax.nn.sigmoid(outs[0:K_CLS])
        reg_c = outs[REG_OFF:REG_OFF + K_REG]

        row0 = c * ROWS_PER_CHUNK
        cls_ref[0, :, pl.ds(row0, ROWS_PER_CHUNK), :] = cls_c.reshape(
            K_CLS, ROWS_PER_CHUNK, W)
        reg_ref[0, :, pl.ds(row0, ROWS_PER_CHUNK), :] = reg_c.reshape(
            K_REG, ROWS_PER_CHUNK, W)


@jax.jit
def kernel(features, W_inter, b_inter, W_cls, b_cls, W_reg, b_reg):
    w_all = jnp.concatenate([
        jnp.transpose(W_inter, (0, 2, 3, 1)).reshape(INTER, 9 * C),
        b_inter[:, None],
        jnp.zeros((INTER, K_BIG - 9 * C - 1), jnp.float32),
    ], axis=1).astype(jnp.bfloat16)
    w_small = jnp.concatenate([
        W_cls.reshape(K_CLS, INTER),
        jnp.zeros((REG_OFF - K_CLS, INTER), jnp.float32),
        W_reg.reshape(K_REG, INTER),
        jnp.zeros((M_SMALL - REG_OFF - K_REG, INTER), jnp.float32),
    ]).astype(jnp.bfloat16)
    bias_cat = jnp.concatenate([
        b_cls,
        jnp.zeros((REG_OFF - K_CLS,), jnp.float32),
        b_reg,
        jnp.zeros((M_SMALL - REG_OFF - K_REG,), jnp.float32),
    ])
    bias_pack = jnp.broadcast_to(bias_cat[:, None], (M_SMALL, W))

    cls, reg = pl.pallas_call(
        _rpn_kernel,
        grid=(B // B_PER_STEP,),
        in_specs=[
            pl.BlockSpec((B_PER_STEP, C, H, W), lambda b: (b, 0, 0, 0)),
            pl.BlockSpec((INTER, K_BIG), lambda b: (0, 0)),
            pl.BlockSpec((M_SMALL, INTER), lambda b: (0, 0)),
            pl.BlockSpec((M_SMALL, W), lambda b: (0, 0)),
        ],
        out_specs=[
            pl.BlockSpec((B_PER_STEP, K_CLS, H, W), lambda b: (b, 0, 0, 0)),
            pl.BlockSpec((B_PER_STEP, K_REG, H, W), lambda b: (b, 0, 0, 0)),
        ],
        out_shape=[
            jax.ShapeDtypeStruct((B, K_CLS, H, W), jnp.float32),
            jax.ShapeDtypeStruct((B, K_REG, H, W), jnp.float32),
        ],
        scratch_shapes=[pltpu.VMEM((K_BIG, HW), jnp.bfloat16)],
        compiler_params=pltpu.CompilerParams(
            dimension_semantics=("arbitrary",)),
    )(features, w_all, w_small, bias_pack)
    return (cls, reg)


# speedup vs baseline: 1.0314x; 1.0314x over previous
import jax
import jax.numpy as jnp
from jax.experimental import pallas as pl
from jax.experimental.pallas import tpu as pltpu

B, C, INTER, H, W = 8, 96, 96, 128, 128
K_CLS, K_REG = 9, 36
HW = H * W
REG_OFF = 16
M_SMALL = 56


def _rpn_kernel(x_ref, w_all_ref, w_small_ref, bias_ref,
                cls_ref, reg_ref, s9_ref):
    w_all = w_all_ref[...]
    w_small = w_small_ref[...]
    b_inter = bias_ref[0:C, 0:1]
    b_small = bias_ref[C:C + M_SMALL, 0:1]
    lane = jax.lax.broadcasted_iota(jnp.int32, (C, HW), 1) & (W - 1)

    x = x_ref[0].astype(jnp.bfloat16).reshape(C, HW)
    xl = jnp.where(lane == 0, 0.0, jnp.roll(x, 1, axis=1))
    xr = jnp.where(lane == W - 1, 0.0, jnp.roll(x, -1, axis=1))
    zrow = jnp.zeros((C, W), jnp.bfloat16)
    for dx, v in ((0, xl), (1, x), (2, xr)):
        r0 = dx * C
        s9_ref[r0:r0 + C, 0:W] = zrow
        s9_ref[r0:r0 + C, pl.ds(W, HW - W)] = v[:, 0:HW - W]
        r1 = (3 + dx) * C
        s9_ref[r1:r1 + C, :] = v
        r2 = (6 + dx) * C
        s9_ref[r2:r2 + C, 0:HW - W] = v[:, W:HW]
        s9_ref[r2:r2 + C, pl.ds(HW - W, W)] = zrow

    acc = jnp.dot(w_all, s9_ref[...],
                  preferred_element_type=jnp.float32) + b_inter
    inter = jnp.maximum(acc, 0.0).astype(jnp.bfloat16)

    outs = jnp.dot(w_small, inter,
                   preferred_element_type=jnp.float32) + b_small
    cls_ref[...] = jax.nn.sigmoid(outs[0:K_CLS]).reshape(1, K_CLS, H, W)
    reg_ref[...] = outs[REG_OFF:REG_OFF + K_REG].reshape(1, K_REG, H, W)


@jax.jit
def kernel(features, W_inter, b_inter, W_cls, b_cls, W_reg, b_reg):
    w_all = jnp.transpose(W_inter, (0, 2, 3, 1)).reshape(
        INTER, 9 * C).astype(jnp.bfloat16)
    w_small = jnp.concatenate([
        W_cls.reshape(K_CLS, INTER),
        jnp.zeros((REG_OFF - K_CLS, INTER), jnp.float32),
        W_reg.reshape(K_REG, INTER),
        jnp.zeros((M_SMALL - REG_OFF - K_REG, INTER), jnp.float32),
    ]).astype(jnp.bfloat16)
    bias_cat = jnp.concatenate([
        b_inter,
        b_cls,
        jnp.zeros((REG_OFF - K_CLS,), jnp.float32),
        b_reg,
        jnp.zeros((M_SMALL - REG_OFF - K_REG,), jnp.float32),
    ])
    bias_pack = jnp.broadcast_to(bias_cat[:, None], (C + M_SMALL, W))

    cls, reg = pl.pallas_call(
        _rpn_kernel,
        grid=(B,),
        in_specs=[
            pl.BlockSpec((1, C, H, W), lambda b: (b, 0, 0, 0)),
            pl.BlockSpec((INTER, 9 * C), lambda b: (0, 0)),
            pl.BlockSpec((M_SMALL, INTER), lambda b: (0, 0)),
            pl.BlockSpec((C + M_SMALL, W), lambda b: (0, 0)),
        ],
        out_specs=[
            pl.BlockSpec((1, K_CLS, H, W), lambda b: (b, 0, 0, 0)),
            pl.BlockSpec((1, K_REG, H, W), lambda b: (b, 0, 0, 0)),
        ],
        out_shape=[
            jax.ShapeDtypeStruct((B, K_CLS, H, W), jnp.float32),
            jax.ShapeDtypeStruct((B, K_REG, H, W), jnp.float32),
        ],
        scratch_shapes=[pltpu.VMEM((9 * C, HW), jnp.bfloat16)],
        compiler_params=pltpu.CompilerParams(
            dimension_semantics=("arbitrary",)),
    )(features, w_all, w_small, bias_pack)
    return (cls, reg)
